# baseline (speedup 1.0000x reference)
import functools

import jax
import jax.numpy as jnp
from jax import lax
from jax.experimental import pallas as pl
from jax.experimental.pallas import tpu as pltpu
from jax.experimental.pallas import tpu_sc as plsc

N = 10000
E = 320000
D = 128

NC = 2
NS = 16
E_PER_SC = E // NC
E_PER_W = E_PER_SC // NS
N_PAD = 10240
RPT = N_PAD // NS

SEG_CHUNK = 88
SEG_ITERS = 114
E_PER_W_PAD = SEG_ITERS * SEG_CHUNK
NBUF = 4
ACC_ROWS = N + 8

HIST_CHUNK = 2000
HIST_ITERS = E_PER_W // HIST_CHUNK
HW = 16

_sc_mesh = plsc.VectorSubcoreMesh(core_axis_name="c", subcore_axis_name="s")
_sc_params = pltpu.CompilerParams(use_tc_tiling_on_sc=False)


@functools.partial(
    pl.kernel,
    out_type=jax.ShapeDtypeStruct((NC, N_PAD, HW), jnp.float32),
    mesh=_sc_mesh,
    scratch_types=[
        pltpu.VMEM((HIST_CHUNK,), jnp.int32),
        pltpu.VMEM((HIST_CHUNK, HW), jnp.float32),
        pltpu.VMEM((RPT, HW), jnp.float32),
        pltpu.VMEM_SHARED((N_PAD, HW), jnp.float32),
    ],
    compiler_params=_sc_params,
)
def _sc_hist(dst_hbm, out_hbm, dst_v, ones_v, zero_v, acc):
    c = lax.axis_index("c")
    s = lax.axis_index("s")

    one16 = jnp.ones((16,), jnp.float32)
    zer16 = jnp.zeros((16,), jnp.float32)

    @pl.loop(0, HIST_CHUNK)
    def _(i):
        ones_v[i, :] = one16

    @pl.loop(0, RPT)
    def _(i):
        zero_v[i, :] = zer16

    pltpu.sync_copy(zero_v, acc.at[pl.ds(s * RPT, RPT)])
    plsc.subcore_barrier()

    base = c * E_PER_SC + s * E_PER_W

    @pl.loop(0, HIST_ITERS)
    def _(i):
        pltpu.sync_copy(dst_hbm.at[pl.ds(base + i * HIST_CHUNK, HIST_CHUNK)],
                        dst_v)
        pltpu.sync_copy(ones_v, acc.at[dst_v], add=True)

    plsc.subcore_barrier()
    row0 = s * RPT
    pltpu.sync_copy(acc.at[pl.ds(row0, RPT)],
                    out_hbm.at[c, pl.ds(row0, RPT)])


@functools.partial(
    pl.kernel,
    out_type=jax.ShapeDtypeStruct((NC, ACC_ROWS, D), jnp.float32),
    mesh=_sc_mesh,
    scratch_types=(
        [pltpu.VMEM((2, SEG_CHUNK), jnp.int32) for _ in range(NBUF)]
        + [pltpu.VMEM((SEG_CHUNK, D), jnp.float32) for _ in range(NBUF)]
        + [pltpu.VMEM_SHARED((ACC_ROWS, D), jnp.float32)]
        + [pltpu.SemaphoreType.DMA for _ in range(3 * NBUF)]
    ),
    compiler_params=_sc_params,
)
def _sc_seg(hs_hbm, eidx_hbm, out_hbm,
            e0, e1, e2, e3, r0, r1, r2, r3, acc,
            i0, i1, i2, i3, g0, g1, g2, g3, s0, s1, s2, s3):
    c = lax.axis_index("c")
    s = lax.axis_index("s")
    w = c * NS + s

    ebuf = (e0, e1, e2, e3)
    rows = (r0, r1, r2, r3)
    isem = (i0, i1, i2, i3)
    gsem = (g0, g1, g2, g3)
    ssem = (s0, s1, s2, s3)

    zer16 = jnp.zeros((16,), jnp.float32)

    @pl.loop(0, SEG_CHUNK)
    def _(i):
        for j in range(D // 16):
            r0[i, pl.ds(j * 16, 16)] = zer16

    zbase = s * (N // NS)
    for k in range(7):
        pltpu.sync_copy(r0.at[pl.ds(0, SEG_CHUNK)],
                        acc.at[pl.ds(zbase + k * SEG_CHUNK, SEG_CHUNK)])
    pltpu.sync_copy(r0.at[pl.ds(0, 9)],
                    acc.at[pl.ds(zbase + 7 * SEG_CHUNK, 9)])
    plsc.subcore_barrier()

    def fetch_idx(j, b):
        pltpu.async_copy(eidx_hbm.at[w, j], ebuf[b], isem[b])

    def wait_idx(b):
        pltpu.make_async_copy(eidx_hbm.at[w, 0], ebuf[b], isem[b]).wait()

    def start_gather(b):
        pltpu.async_copy(hs_hbm.at[ebuf[b].at[0]], rows[b], gsem[b])

    def wait_gather(b):
        pltpu.make_async_copy(hs_hbm.at[ebuf[b].at[0]], rows[b],
                              gsem[b]).wait()

    def start_scatter(b):
        pltpu.async_copy(rows[b], acc.at[ebuf[b].at[1]], ssem[b], add=True)

    def wait_scatter(b):
        pltpu.make_async_copy(rows[b], acc.at[ebuf[b].at[1]], ssem[b]).wait()

    @pl.loop(0, (SEG_ITERS + 2) // NBUF)
    def _(t):
        for b in range(NBUF):
            i = NBUF * t + b
            sb = (b + 2) % NBUF
            gb = (b + 3) % NBUF

            @pl.when((i >= 2) & (i < SEG_ITERS + 2))
            def _():
                wait_gather(sb)
                start_scatter(sb)

            @pl.when((i >= 1) & (i < SEG_ITERS + 1))
            def _():
                wait_idx(gb)
                start_gather(gb)

            @pl.when((i >= NBUF) & (i < SEG_ITERS))
            def _():
                wait_scatter(b)

            @pl.when(i < SEG_ITERS)
            def _():
                fetch_idx(i, b)

    for b in ((SEG_ITERS - 4) % NBUF, (SEG_ITERS - 3) % NBUF,
              (SEG_ITERS - 2) % NBUF, (SEG_ITERS - 1) % NBUF):
        wait_scatter(b)

    plsc.subcore_barrier()
    row0 = s * (N // NS)
    pltpu.sync_copy(acc.at[pl.ds(row0, N // NS)],
                    out_hbm.at[c, pl.ds(row0, N // NS)])


_BR = 1000
_GRID = N // _BR


def _prep_body(x_ref, w1_ref, d0_ref, d1_ref, hs_ref, dinvb_ref):
    deg = d0_ref[:, 0:1] + d1_ref[:, 0:1] + 1.0
    dinv = lax.rsqrt(deg)
    dinvb = jnp.broadcast_to(dinv, (_BR, D))
    h1 = jnp.dot(x_ref[...], w1_ref[...], preferred_element_type=jnp.float32)
    hs_ref[...] = h1 * dinvb
    dinvb_ref[...] = dinvb


def _mid_body(sa_ref, sb_ref, hs_ref, dinvb_ref, w2_ref, b1_ref,
              hs2_ref):
    dinvb = dinvb_ref[...]
    h = dinvb * (sa_ref[...] + sb_ref[...] + hs_ref[...]) + b1_ref[...]
    h = jnp.maximum(h, 0.0)
    h2 = jnp.dot(h, w2_ref[...], preferred_element_type=jnp.float32)
    hs2_ref[...] = h2 * dinvb


def _final_body(sa_ref, sb_ref, hs2_ref, dinvb_ref, b2_ref, out_ref):
    o = dinvb_ref[...] * (sa_ref[...] + sb_ref[...] + hs2_ref[...]) + b2_ref[...]
    m = jnp.max(o, axis=1, keepdims=True)
    z = o - m
    lse = jnp.log(jnp.sum(jnp.exp(z), axis=1, keepdims=True))
    out_ref[...] = z - lse


def _row_spec(w):
    return pl.BlockSpec((_BR, w), lambda i: (i, 0))


def _full_spec(h, w):
    return pl.BlockSpec((h, w), lambda i: (0, 0))


_prep = pl.pallas_call(
    _prep_body,
    grid=(_GRID,),
    in_specs=[_row_spec(D), _full_spec(D, D), _row_spec(HW), _row_spec(HW)],
    out_specs=[_row_spec(D), _row_spec(D)],
    out_shape=[jax.ShapeDtypeStruct((N, D), jnp.float32),
               jax.ShapeDtypeStruct((N, D), jnp.float32)],
)

_mid = pl.pallas_call(
    _mid_body,
    grid=(_GRID,),
    in_specs=[_row_spec(D), _row_spec(D), _row_spec(D), _row_spec(D),
              _full_spec(D, D), _full_spec(1, D)],
    out_specs=_row_spec(D),
    out_shape=jax.ShapeDtypeStruct((N, D), jnp.float32),
)

_final = pl.pallas_call(
    _final_body,
    grid=(_GRID,),
    in_specs=[_row_spec(D), _row_spec(D), _row_spec(D), _row_spec(D),
              _full_spec(1, D)],
    out_specs=_row_spec(D),
    out_shape=jax.ShapeDtypeStruct((N, D), jnp.float32),
)


@jax.jit
def kernel(x, edge_index, W1, b1, W2, b2):
    src = edge_index[0]
    dst = edge_index[1]

    nw = NC * NS
    pad = E_PER_W_PAD - E_PER_W
    src2 = jnp.pad(src.reshape(nw, E_PER_W), ((0, 0), (0, pad)))
    dst2 = jnp.pad(dst.reshape(nw, E_PER_W), ((0, 0), (0, pad)),
                   constant_values=N)
    eidx = jnp.stack(
        [src2.reshape(nw, SEG_ITERS, SEG_CHUNK),
         dst2.reshape(nw, SEG_ITERS, SEG_CHUNK)], axis=2)

    degp = _sc_hist(dst)
    hs1, dinvb = _prep(x, W1, degp[0], degp[1])

    seg1 = _sc_seg(hs1, eidx)
    hs2 = _mid(seg1[0], seg1[1], hs1, dinvb, W2, b1.reshape(1, D))

    seg2 = _sc_seg(hs2, eidx)
    return _final(seg2[0], seg2[1], hs2, dinvb, b2.reshape(1, D))
